# baseline (speedup 1.0000x reference)
import functools

import jax
import jax.numpy as jnp
from jax import lax
from jax.experimental import pallas as pl
from jax.experimental.pallas import tpu as pltpu
from jax.experimental.pallas import tpu_sc as plsc

N_ATOMS = 10000
MAX_B = 32
E = 320000
H = 128
NC = 2
NS = 16
NW = NC * NS
NLANE = 16

APW = 320
N_PAD = NW * APW
CA = 4
EPW = E // NW
CE = 80

_mesh = plsc.VectorSubcoreMesh(core_axis_name="c", subcore_axis_name="s")


def _worker_id():
    return lax.axis_index("s") * NC + lax.axis_index("c")


@functools.partial(
    pl.kernel,
    out_type=jax.ShapeDtypeStruct((N_PAD, H), jnp.float32),
    mesh=_mesh,
    scratch_types=[
        pltpu.VMEM((CA * MAX_B,), jnp.int32),
        pltpu.VMEM((CA * MAX_B, H), jnp.float32),
        pltpu.VMEM((CA, H), jnp.float32),
        pltpu.SemaphoreType.DMA,
    ],
)
def _segsum_relu(msg_hbm, a2b_hbm, out_hbm, idx_v, rows_v, acc_v, sem):
    wid = _worker_id()

    def chunk_body(c, carry):
        base = wid * APW + c * CA
        pltpu.sync_copy(a2b_hbm.at[pl.ds(base * MAX_B, CA * MAX_B)], idx_v)
        pltpu.async_copy(msg_hbm.at[idx_v], rows_v, sem).wait()

        def atom_body(a, carry2):
            def nb_body(k, accs):
                r = a * MAX_B + k
                return tuple(
                    accs[s]
                    + jnp.maximum(rows_v[r, pl.ds(s * NLANE, NLANE)], 0.0)
                    for s in range(H // NLANE)
                )

            zeros = tuple(
                jnp.zeros((NLANE,), jnp.float32) for _ in range(H // NLANE)
            )
            accs = lax.fori_loop(0, MAX_B, nb_body, zeros)
            for s in range(H // NLANE):
                acc_v[a, pl.ds(s * NLANE, NLANE)] = accs[s]
            return carry2

        lax.fori_loop(0, CA, atom_body, 0)
        pltpu.sync_copy(acc_v, out_hbm.at[pl.ds(base, CA)])
        return carry

    lax.fori_loop(0, APW // CA, chunk_body, 0)


@functools.partial(
    pl.kernel,
    out_type=jax.ShapeDtypeStruct((E, H), jnp.float32),
    mesh=_mesh,
    scratch_types=[
        pltpu.VMEM((CE,), jnp.int32),
        pltpu.VMEM((CE,), jnp.int32),
        pltpu.VMEM((CE, H), jnp.float32),
        pltpu.VMEM((CE, H), jnp.float32),
        pltpu.SemaphoreType.DMA,
        pltpu.SemaphoreType.DMA,
    ],
)
def _edge_delta(amsg_hbm, msg_hbm, b2a_hbm, b2revb_hbm, t_hbm,
                idxa_v, idxm_v, arows_v, mrows_v, sema, semm):
    wid = _worker_id()

    def chunk_body(c, carry):
        base = wid * EPW + c * CE
        pltpu.sync_copy(b2a_hbm.at[pl.ds(base, CE)], idxa_v)
        pltpu.sync_copy(b2revb_hbm.at[pl.ds(base, CE)], idxm_v)
        cpa = pltpu.async_copy(amsg_hbm.at[idxa_v], arows_v, sema)
        cpm = pltpu.async_copy(msg_hbm.at[idxm_v], mrows_v, semm)
        cpa.wait()
        cpm.wait()

        def edge_body(e, carry2):
            for s in range(H // NLANE):
                sl = pl.ds(s * NLANE, NLANE)
                arows_v[e, sl] = arows_v[e, sl] - jnp.maximum(
                    mrows_v[e, sl], 0.0)
            return carry2

        lax.fori_loop(0, CE, edge_body, 0)
        pltpu.sync_copy(arows_v, t_hbm.at[pl.ds(base, CE)])
        return carry

    lax.fori_loop(0, EPW // CE, chunk_body, 0)


def _mm_in(f_bonds, W_i):
    BE = 1000

    def body(fb_ref, wi_ref, out_ref):
        out_ref[...] = jnp.dot(fb_ref[...], wi_ref[...],
                               preferred_element_type=jnp.float32)

    return pl.pallas_call(
        body,
        grid=(E // BE,),
        in_specs=[
            pl.BlockSpec((BE, f_bonds.shape[1]), lambda i: (i, 0)),
            pl.BlockSpec(W_i.shape, lambda i: (0, 0)),
        ],
        out_specs=pl.BlockSpec((BE, H), lambda i: (i, 0)),
        out_shape=jax.ShapeDtypeStruct((E, H), jnp.float32),
    )(f_bonds, W_i)


def _mm_update(inp, t, W_h):
    BE = 1000

    def body(inp_ref, t_ref, wh_ref, out_ref):
        out_ref[...] = inp_ref[...] + jnp.dot(
            t_ref[...], wh_ref[...], preferred_element_type=jnp.float32)

    return pl.pallas_call(
        body,
        grid=(E // BE,),
        in_specs=[
            pl.BlockSpec((BE, H), lambda i: (i, 0)),
            pl.BlockSpec((BE, H), lambda i: (i, 0)),
            pl.BlockSpec((H, H), lambda i: (0, 0)),
        ],
        out_specs=pl.BlockSpec((BE, H), lambda i: (i, 0)),
        out_shape=jax.ShapeDtypeStruct((E, H), jnp.float32),
    )(inp, t, W_h)


def _readout(f_atoms, a_message, W_o, b_o, W1, b1, W2, b2, W3, b3):
    BA = 2000
    Wo_a = W_o[:H]
    Wo_m = W_o[H:]
    W3p = jnp.zeros((H, H), jnp.float32).at[:, :W3.shape[1]].set(W3)
    b3p = jnp.zeros((1, H), jnp.float32).at[0, :b3.shape[0]].set(b3)

    def body(fa_ref, am_ref, woa_ref, wom_ref, bo_ref, w1_ref, b1_ref,
             w2_ref, b2_ref, w3_ref, b3_ref, out_ref):
        ah = jax.nn.relu(
            jnp.dot(fa_ref[...], woa_ref[...],
                    preferred_element_type=jnp.float32)
            + jnp.dot(am_ref[...], wom_ref[...],
                      preferred_element_type=jnp.float32)
            + bo_ref[...])
        h = jax.nn.relu(jnp.dot(ah, w1_ref[...],
                                preferred_element_type=jnp.float32)
                        + b1_ref[...])
        h = jax.nn.relu(jnp.dot(h, w2_ref[...],
                                preferred_element_type=jnp.float32)
                        + b2_ref[...])
        out_ref[...] = jax.nn.sigmoid(
            jnp.dot(h, w3_ref[...], preferred_element_type=jnp.float32)
            + b3_ref[...])

    full = lambda shape: pl.BlockSpec(shape, lambda i: (0, 0))
    out = pl.pallas_call(
        body,
        grid=(N_ATOMS // BA,),
        in_specs=[
            pl.BlockSpec((BA, H), lambda i: (i, 0)),
            pl.BlockSpec((BA, H), lambda i: (i, 0)),
            full((H, H)), full((H, H)), full((1, H)),
            full((H, H)), full((1, H)),
            full((H, H)), full((1, H)),
            full((H, H)), full((1, H)),
        ],
        out_specs=pl.BlockSpec((BA, H), lambda i: (i, 0)),
        out_shape=jax.ShapeDtypeStruct((N_ATOMS, H), jnp.float32),
    )(f_atoms, a_message, Wo_a, Wo_m, b_o.reshape(1, H),
      W1, b1.reshape(1, H), W2, b2.reshape(1, H), W3p, b3p)
    return out


def kernel(f_atoms, f_bonds, a2b, b2a, b2revb, W_i, W_h, W_o, b_o,
           W1, b1, W2, b2, W3, b3):
    a2b_flat = jnp.zeros((N_PAD, MAX_B), jnp.int32).at[:N_ATOMS].set(
        a2b).reshape(-1)

    inp = _mm_in(f_bonds, W_i)
    msg_pre = inp
    for _ in range(2):
        a_msg = _segsum_relu(msg_pre, a2b_flat)
        t = _edge_delta(a_msg, msg_pre, b2a, b2revb)
        msg_pre = _mm_update(inp, t, W_h)
    a_message = _segsum_relu(msg_pre, a2b_flat)[:N_ATOMS]
    out = _readout(f_atoms, a_message, W_o, b_o, W1, b1, W2, b2, W3, b3)
    return out[1:, :1]
